# baseline (speedup 1.0000x reference)
import functools

import jax
import jax.numpy as jnp
from jax import lax
from jax.experimental import pallas as pl
from jax.experimental.pallas import tpu as pltpu
from jax.experimental.pallas import tpu_sc as plsc

B, S, D, F, E, TOPK = 1, 2048, 2048, 1024, 8, 2
EPS = 1e-05

T = S
TB = 128
NBLK = T * TOPK // TB + E
NSLOT = NBLK * TB
NORM_TB = 256

NW = 32
DISP_CHUNK = 16
COMB_CHUNK = 8


def _norm_router_kernel(x_ref, ln1_ref, ln2_ref, rw_ref, h1_ref, xn_ref,
                        logits_ref):
    x = x_ref[...]
    v1 = jnp.mean(jnp.square(x), axis=-1, keepdims=True)
    h1 = x + x * lax.rsqrt(v1 + EPS) * ln1_ref[...]
    v2 = jnp.mean(jnp.square(h1), axis=-1, keepdims=True)
    xn = h1 * lax.rsqrt(v2 + EPS) * ln2_ref[...]
    h1_ref[...] = h1
    xn_ref[...] = xn
    logits_ref[...] = jnp.dot(xn, rw_ref[...],
                              preferred_element_type=jnp.float32)


RC = 512
PC = 512


def _top2(logits):
    n = logits.shape[0]
    lanes = lax.broadcasted_iota(jnp.int32, (n, E), 1)
    m1 = jnp.max(logits, axis=1, keepdims=True)
    i1 = jnp.min(jnp.where(logits == m1, lanes, E), axis=1, keepdims=True)
    l2 = jnp.where(lanes == i1, -jnp.inf, logits)
    m2 = jnp.max(l2, axis=1, keepdims=True)
    i2 = jnp.min(jnp.where(l2 == m2, lanes, E), axis=1, keepdims=True)
    b = jnp.exp(m2 - m1)
    p1 = 1.0 / (1.0 + b)
    p2 = b / (1.0 + b)
    oh1 = (lanes == i1).astype(jnp.float32)
    oh2 = (lanes == i2).astype(jnp.float32)
    return oh1, oh2, p1, p2


def _expert_layout(oh1, oh2):
    c1 = jnp.sum(oh1, axis=0, keepdims=True)
    cnt_i = (c1 + jnp.sum(oh2, axis=0, keepdims=True)).astype(jnp.int32)
    nblk = (cnt_i + (TB - 1)) // TB
    padded = (nblk * TB).astype(jnp.float32)
    er = lax.broadcasted_iota(jnp.int32, (E, E), 0)
    ec = lax.broadcasted_iota(jnp.int32, (E, E), 1)
    ltri = (er < ec).astype(jnp.float32)
    base = jnp.dot(padded, ltri, preferred_element_type=jnp.float32)
    return c1, nblk, base


def _route_dst_kernel(logits_ref, logch_ref, dst1_ref, dst2_ref, bexp_ref):
    ci = pl.program_id(0)
    logits = logits_ref[...]
    oh1, oh2, _, _ = _top2(logits)
    c1, nblk, base = _expert_layout(oh1, oh2)

    rid = lax.broadcasted_iota(jnp.int32, (RC, T), 0) + ci * RC
    cols = lax.broadcasted_iota(jnp.int32, (RC, T), 1)
    strict = (cols < rid).astype(jnp.float32)
    cum1 = jnp.dot(strict, oh1, preferred_element_type=jnp.float32)
    cum2 = jnp.dot(strict, oh2, preferred_element_type=jnp.float32)

    oh1c, oh2c, _, _ = _top2(logch_ref[...])
    r1 = jnp.sum(cum1 * oh1c, axis=1, keepdims=True)
    r2 = jnp.sum(cum2 * oh2c, axis=1, keepdims=True)
    base1 = jnp.sum(oh1c * base, axis=1, keepdims=True)
    base2 = jnp.sum(oh2c * base, axis=1, keepdims=True)
    c1_at2 = jnp.sum(oh2c * c1, axis=1, keepdims=True)
    dst1 = base1 + r1
    dst2 = base2 + c1_at2 + r2
    dst1_ref[...] = dst1.astype(jnp.int32).reshape(1, RC, 1)
    dst2_ref[...] = dst2.astype(jnp.int32).reshape(1, RC, 1)

    endblk = (base / TB).astype(jnp.int32) + nblk
    bid = lax.broadcasted_iota(jnp.int32, (E, NBLK), 1)
    done = (bid >= endblk.reshape(E, 1)).astype(jnp.int32)
    bexp_ref[...] = jnp.minimum(
        jnp.sum(done, axis=0, keepdims=True), E - 1).reshape(1, 1, NBLK)


def _route_perm_kernel(logits_ref, dst1_ref, dst2_ref, perm_ref, wslot_ref):
    pc = pl.program_id(0)
    _, _, p1, p2 = _top2(logits_ref[...])
    dst1 = dst1_ref[...].reshape(T, 1).astype(jnp.float32)
    dst2 = dst2_ref[...].reshape(T, 1).astype(jnp.float32)
    tok = lax.broadcasted_iota(jnp.int32, (T, 1), 0).astype(jnp.float32)
    pv = (lax.broadcasted_iota(jnp.int32, (1, PC), 1)
          + pc * PC).astype(jnp.float32)
    msk1 = (dst1 == pv).astype(jnp.float32)
    msk2 = (dst2 == pv).astype(jnp.float32)
    permc = jnp.sum(msk1 * tok + msk2 * tok, axis=0, keepdims=True)
    wc = jnp.sum(msk1 * p1 + msk2 * p2, axis=0, keepdims=True)
    perm_ref[...] = permc.astype(jnp.int32).reshape(1, 1, PC)
    wslot_ref[...] = wc.reshape(1, 1, PC)


def _dispatch_body(xn_hbm, perm_hbm, buf_hbm, idx_v, rows_a, rows_b,
                   sem_a, sem_b):
    wid = lax.axis_index("s") * 2 + lax.axis_index("c")
    per_w = NSLOT // NW
    nch = per_w // DISP_CHUNK
    base = wid * per_w
    pltpu.sync_copy(perm_hbm.at[pl.ds(base, per_w)], idx_v)
    bufs = (rows_a, rows_b)
    sems = (sem_a, sem_b)
    pend = pltpu.async_copy(
        xn_hbm.at[idx_v.at[pl.ds(0, DISP_CHUNK)]], rows_a, sem_a)
    for c in range(nch):
        cur = pend
        if c + 1 < nch:
            pend = pltpu.async_copy(
                xn_hbm.at[idx_v.at[pl.ds((c + 1) * DISP_CHUNK, DISP_CHUNK)]],
                bufs[(c + 1) % 2], sems[(c + 1) % 2])
        cur.wait()
        pltpu.sync_copy(bufs[c % 2],
                        buf_hbm.at[pl.ds(base + c * DISP_CHUNK, DISP_CHUNK)])


def _ffn_up_kernel(bexp_ref, xb_ref, wg_ref, wu_ref, act_ref):
    del bexp_ref
    x = xb_ref[...].astype(jnp.bfloat16)
    gate = jnp.dot(x, wg_ref[0], preferred_element_type=jnp.float32)
    up = jnp.dot(x, wu_ref[0], preferred_element_type=jnp.float32)
    act = gate * lax.logistic(gate) * up
    act_ref[...] = act.astype(jnp.bfloat16)


def _ffn_down_kernel(bexp_ref, act_ref, ws_ref, wd_ref, out_ref):
    del bexp_ref
    y = jnp.dot(act_ref[...], wd_ref[0], preferred_element_type=jnp.float32)
    out_ref[...] = y * ws_ref[...]


def _combine_body(h1_hbm, ybuf_hbm, dst1_hbm, dst2_hbm, out_hbm,
                  i1_v, i2_v, h_a, h_b, r1_a, r1_b, r2_a, r2_b,
                  sh_a, sh_b, s1_a, s1_b, s2_a, s2_b):
    wid = lax.axis_index("s") * 2 + lax.axis_index("c")
    per_w = T // NW
    nch = per_w // COMB_CHUNK
    tbase = wid * per_w
    pltpu.sync_copy(dst1_hbm.at[pl.ds(tbase, per_w)], i1_v)
    pltpu.sync_copy(dst2_hbm.at[pl.ds(tbase, per_w)], i2_v)
    hs = (h_a, h_b)
    r1s = (r1_a, r1_b)
    r2s = (r2_a, r2_b)
    sems = ((sh_a, s1_a, s2_a), (sh_b, s1_b, s2_b))

    def start(c):
        p = c % 2
        t0 = tbase + c * COMB_CHUNK
        isl = pl.ds(c * COMB_CHUNK, COMB_CHUNK)
        return (
            pltpu.async_copy(h1_hbm.at[pl.ds(t0, COMB_CHUNK)], hs[p],
                             sems[p][0]),
            pltpu.async_copy(ybuf_hbm.at[i1_v.at[isl]], r1s[p], sems[p][1]),
            pltpu.async_copy(ybuf_hbm.at[i2_v.at[isl]], r2s[p], sems[p][2]),
        )

    pend = start(0)
    for c in range(nch):
        p = c % 2
        cur = pend
        if c + 1 < nch:
            pend = start(c + 1)
        for cp in cur:
            cp.wait()
        for i in range(COMB_CHUNK):
            def body(k, _):
                sl = pl.ds(k * 16, 16)
                hs[p][i, sl] = hs[p][i, sl] + r1s[p][i, sl] + r2s[p][i, sl]
                return 0
            lax.fori_loop(0, D // 16, body, 0)
        pltpu.sync_copy(hs[p],
                        out_hbm.at[pl.ds(tbase + c * COMB_CHUNK, COMB_CHUNK)])


@jax.jit
def kernel(hidden_states, ln1_w, ln2_w, router_w, w_gate, w_up, w_down):
    x = hidden_states.reshape(T, D)
    ln1 = ln1_w.reshape(1, D)
    ln2 = ln2_w.reshape(1, D)

    h1, xn, logits = pl.pallas_call(
        _norm_router_kernel,
        grid=(T // NORM_TB,),
        in_specs=[
            pl.BlockSpec((NORM_TB, D), lambda i: (i, 0)),
            pl.BlockSpec((1, D), lambda i: (0, 0)),
            pl.BlockSpec((1, D), lambda i: (0, 0)),
            pl.BlockSpec((D, E), lambda i: (0, 0)),
        ],
        out_specs=[
            pl.BlockSpec((NORM_TB, D), lambda i: (i, 0)),
            pl.BlockSpec((NORM_TB, D), lambda i: (i, 0)),
            pl.BlockSpec((NORM_TB, E), lambda i: (i, 0)),
        ],
        out_shape=[
            jax.ShapeDtypeStruct((T, D), jnp.float32),
            jax.ShapeDtypeStruct((T, D), jnp.float32),
            jax.ShapeDtypeStruct((T, E), jnp.float32),
        ],
    )(x, ln1, ln2, router_w)

    dst1_3d, dst2_3d, bexp = pl.pallas_call(
        _route_dst_kernel,
        grid=(T // RC,),
        in_specs=[
            pl.BlockSpec((T, E), lambda i: (0, 0)),
            pl.BlockSpec((RC, E), lambda i: (i, 0)),
        ],
        out_specs=[
            pl.BlockSpec((1, RC, 1), lambda i: (i, 0, 0)),
            pl.BlockSpec((1, RC, 1), lambda i: (i, 0, 0)),
            pl.BlockSpec((1, 1, NBLK), lambda i: (0, 0, 0)),
        ],
        out_shape=[
            jax.ShapeDtypeStruct((T // RC, RC, 1), jnp.int32),
            jax.ShapeDtypeStruct((T // RC, RC, 1), jnp.int32),
            jax.ShapeDtypeStruct((1, 1, NBLK), jnp.int32),
        ],
    )(logits, logits)

    perm_3d, wslot_3d = pl.pallas_call(
        _route_perm_kernel,
        grid=(NSLOT // PC,),
        in_specs=[
            pl.BlockSpec((T, E), lambda i: (0, 0)),
            pl.BlockSpec((T // RC, RC, 1), lambda i: (0, 0, 0)),
            pl.BlockSpec((T // RC, RC, 1), lambda i: (0, 0, 0)),
        ],
        out_specs=[
            pl.BlockSpec((1, 1, PC), lambda i: (i, 0, 0)),
            pl.BlockSpec((1, 1, PC), lambda i: (i, 0, 0)),
        ],
        out_shape=[
            jax.ShapeDtypeStruct((NSLOT // PC, 1, PC), jnp.int32),
            jax.ShapeDtypeStruct((NSLOT // PC, 1, PC), jnp.float32),
        ],
    )(logits, dst1_3d, dst2_3d)

    perm = perm_3d.reshape(NSLOT)
    dst1 = dst1_3d.reshape(T)
    dst2 = dst2_3d.reshape(T)
    bexp = bexp.reshape(NBLK)
    wslot2d = wslot_3d.reshape(NSLOT, 1)

    mesh = plsc.VectorSubcoreMesh(core_axis_name="c", subcore_axis_name="s")
    xbuf = pl.kernel(
        _dispatch_body,
        mesh=mesh,
        out_type=jax.ShapeDtypeStruct((NSLOT, D), jnp.float32),
        scratch_types=[
            pltpu.VMEM((NSLOT // NW,), jnp.int32),
            pltpu.VMEM((DISP_CHUNK, D), jnp.float32),
            pltpu.VMEM((DISP_CHUNK, D), jnp.float32),
            pltpu.SemaphoreType.DMA,
            pltpu.SemaphoreType.DMA,
        ],
    )(xn, perm)

    act = pl.pallas_call(
        _ffn_up_kernel,
        grid_spec=pltpu.PrefetchScalarGridSpec(
            num_scalar_prefetch=1,
            grid=(NBLK,),
            in_specs=[
                pl.BlockSpec((TB, D), lambda b, be: (b, 0)),
                pl.BlockSpec((1, D, F), lambda b, be: (be[b], 0, 0)),
                pl.BlockSpec((1, D, F), lambda b, be: (be[b], 0, 0)),
            ],
            out_specs=pl.BlockSpec((TB, F), lambda b, be: (b, 0)),
        ),
        out_shape=jax.ShapeDtypeStruct((NSLOT, F), jnp.bfloat16),
        compiler_params=pltpu.CompilerParams(
            vmem_limit_bytes=60 * 1024 * 1024),
    )(bexp, xbuf, w_gate.astype(jnp.bfloat16), w_up.astype(jnp.bfloat16))

    ybuf = pl.pallas_call(
        _ffn_down_kernel,
        grid_spec=pltpu.PrefetchScalarGridSpec(
            num_scalar_prefetch=1,
            grid=(NBLK,),
            in_specs=[
                pl.BlockSpec((TB, F), lambda b, be: (b, 0)),
                pl.BlockSpec((TB, 1), lambda b, be: (b, 0)),
                pl.BlockSpec((1, F, D), lambda b, be: (be[b], 0, 0)),
            ],
            out_specs=pl.BlockSpec((TB, D), lambda b, be: (b, 0)),
        ),
        out_shape=jax.ShapeDtypeStruct((NSLOT, D), jnp.float32),
        compiler_params=pltpu.CompilerParams(
            vmem_limit_bytes=60 * 1024 * 1024),
    )(bexp, act, wslot2d, w_down.astype(jnp.bfloat16))

    out = pl.kernel(
        _combine_body,
        mesh=mesh,
        out_type=jax.ShapeDtypeStruct((T, D), jnp.float32),
        scratch_types=(
            [pltpu.VMEM((T // NW,), jnp.int32)] * 2
            + [pltpu.VMEM((COMB_CHUNK, D), jnp.float32)] * 6
            + [pltpu.SemaphoreType.DMA] * 6
        ),
    )(h1, ybuf, dst1, dst2)

    return out.reshape(B, S, D)


# speedup vs baseline: 1.1264x; 1.1264x over previous
import functools

import jax
import jax.numpy as jnp
from jax import lax
from jax.experimental import pallas as pl
from jax.experimental.pallas import tpu as pltpu
from jax.experimental.pallas import tpu_sc as plsc

B, S, D, F, E, TOPK = 1, 2048, 2048, 1024, 8, 2
EPS = 1e-05

T = S
TB = 128
NBLK = T * TOPK // TB + E
NSLOT = NBLK * TB
NORM_TB = 256

NW = 32
DISP_CHUNK = 24
COMB_CHUNK = 8


def _norm_router_kernel(x_ref, ln1_ref, ln2_ref, rw_ref, h1_ref, xn_ref,
                        logits_ref):
    x = x_ref[...]
    v1 = jnp.mean(jnp.square(x), axis=-1, keepdims=True)
    h1 = x + x * lax.rsqrt(v1 + EPS) * ln1_ref[...]
    v2 = jnp.mean(jnp.square(h1), axis=-1, keepdims=True)
    xn = h1 * lax.rsqrt(v2 + EPS) * ln2_ref[...]
    h1_ref[...] = h1
    xn_ref[...] = xn
    logits_ref[...] = jnp.dot(xn, rw_ref[...],
                              preferred_element_type=jnp.float32)


RC = 512
PC = 512


def _top2(logits):
    n = logits.shape[0]
    lanes = lax.broadcasted_iota(jnp.int32, (n, E), 1)
    m1 = jnp.max(logits, axis=1, keepdims=True)
    i1 = jnp.min(jnp.where(logits == m1, lanes, E), axis=1, keepdims=True)
    l2 = jnp.where(lanes == i1, -jnp.inf, logits)
    m2 = jnp.max(l2, axis=1, keepdims=True)
    i2 = jnp.min(jnp.where(l2 == m2, lanes, E), axis=1, keepdims=True)
    b = jnp.exp(m2 - m1)
    p1 = 1.0 / (1.0 + b)
    p2 = b / (1.0 + b)
    oh1 = (lanes == i1).astype(jnp.float32)
    oh2 = (lanes == i2).astype(jnp.float32)
    return oh1, oh2, p1, p2


def _expert_layout(oh1, oh2):
    c1 = jnp.sum(oh1, axis=0, keepdims=True)
    cnt_i = (c1 + jnp.sum(oh2, axis=0, keepdims=True)).astype(jnp.int32)
    nblk = (cnt_i + (TB - 1)) // TB
    padded = (nblk * TB).astype(jnp.float32)
    er = lax.broadcasted_iota(jnp.int32, (E, E), 0)
    ec = lax.broadcasted_iota(jnp.int32, (E, E), 1)
    ltri = (er < ec).astype(jnp.float32)
    base = jnp.dot(padded, ltri, preferred_element_type=jnp.float32)
    return c1, nblk, base


def _route_dst_kernel(logits_ref, logch_ref, dst1_ref, dst2_ref, bexp_ref):
    ci = pl.program_id(0)
    logits = logits_ref[...]
    oh1, oh2, _, _ = _top2(logits)
    c1, nblk, base = _expert_layout(oh1, oh2)

    rid = lax.broadcasted_iota(jnp.int32, (RC, T), 0) + ci * RC
    cols = lax.broadcasted_iota(jnp.int32, (RC, T), 1)
    strict = (cols < rid).astype(jnp.float32)
    cum1 = jnp.dot(strict, oh1, preferred_element_type=jnp.float32)
    cum2 = jnp.dot(strict, oh2, preferred_element_type=jnp.float32)

    oh1c, oh2c, _, _ = _top2(logch_ref[...])
    r1 = jnp.sum(cum1 * oh1c, axis=1, keepdims=True)
    r2 = jnp.sum(cum2 * oh2c, axis=1, keepdims=True)
    base1 = jnp.sum(oh1c * base, axis=1, keepdims=True)
    base2 = jnp.sum(oh2c * base, axis=1, keepdims=True)
    c1_at2 = jnp.sum(oh2c * c1, axis=1, keepdims=True)
    dst1 = base1 + r1
    dst2 = base2 + c1_at2 + r2
    dst1_ref[...] = dst1.astype(jnp.int32).reshape(1, RC, 1)
    dst2_ref[...] = dst2.astype(jnp.int32).reshape(1, RC, 1)

    endblk = (base / TB).astype(jnp.int32) + nblk
    bid = lax.broadcasted_iota(jnp.int32, (E, NBLK), 1)
    done = (bid >= endblk.reshape(E, 1)).astype(jnp.int32)
    bexp_ref[...] = jnp.minimum(
        jnp.sum(done, axis=0, keepdims=True), E - 1).reshape(1, 1, NBLK)


def _route_perm_kernel(logits_ref, dst1_ref, dst2_ref, perm_ref, wslot_ref):
    pc = pl.program_id(0)
    _, _, p1, p2 = _top2(logits_ref[...])
    dst1 = dst1_ref[...].reshape(T, 1).astype(jnp.float32)
    dst2 = dst2_ref[...].reshape(T, 1).astype(jnp.float32)
    tok = lax.broadcasted_iota(jnp.int32, (T, 1), 0).astype(jnp.float32)
    pv = (lax.broadcasted_iota(jnp.int32, (1, PC), 1)
          + pc * PC).astype(jnp.float32)
    msk1 = (dst1 == pv).astype(jnp.float32)
    msk2 = (dst2 == pv).astype(jnp.float32)
    permc = jnp.sum(msk1 * tok + msk2 * tok, axis=0, keepdims=True)
    wc = jnp.sum(msk1 * p1 + msk2 * p2, axis=0, keepdims=True)
    perm_ref[...] = permc.astype(jnp.int32).reshape(1, 1, PC)
    wslot_ref[...] = wc.reshape(1, 1, PC)


def _dispatch_body(xn_hbm, perm_hbm, buf_hbm, idx_v, rows_a, rows_b,
                   sem_a, sem_b):
    wid = lax.axis_index("s") * 2 + lax.axis_index("c")
    per_w = NSLOT // NW
    base = wid * per_w
    pltpu.sync_copy(perm_hbm.at[pl.ds(base, per_w)], idx_v)
    bufs = (rows_a, rows_b)
    sems = (sem_a, sem_b)
    offs = list(range(0, per_w - DISP_CHUNK + 1, DISP_CHUNK))
    if offs[-1] != per_w - DISP_CHUNK:
        offs.append(per_w - DISP_CHUNK)
    pend = pltpu.async_copy(
        xn_hbm.at[idx_v.at[pl.ds(offs[0], DISP_CHUNK)]], rows_a, sem_a)
    for c in range(len(offs)):
        cur = pend
        if c + 1 < len(offs):
            pend = pltpu.async_copy(
                xn_hbm.at[idx_v.at[pl.ds(offs[c + 1], DISP_CHUNK)]],
                bufs[(c + 1) % 2], sems[(c + 1) % 2])
        cur.wait()
        pltpu.sync_copy(bufs[c % 2],
                        buf_hbm.at[pl.ds(base + offs[c], DISP_CHUNK)])


def _ffn_up_kernel(bexp_ref, xb_ref, wg_ref, wu_ref, act_ref):
    del bexp_ref
    x = xb_ref[...]
    gate = jnp.dot(x, wg_ref[0], preferred_element_type=jnp.float32)
    up = jnp.dot(x, wu_ref[0], preferred_element_type=jnp.float32)
    act_ref[...] = gate * lax.logistic(gate) * up


def _ffn_down_kernel(bexp_ref, act_ref, ws_ref, wd_ref, out_ref):
    del bexp_ref
    y = jnp.dot(act_ref[...], wd_ref[0], preferred_element_type=jnp.float32)
    out_ref[...] = y * ws_ref[...]


def _combine_body(h1_hbm, ybuf_hbm, dst1_hbm, dst2_hbm, out_hbm,
                  i1_v, i2_v, h_a, h_b, r1_a, r1_b, r2_a, r2_b,
                  sh_a, sh_b, s1_a, s1_b, s2_a, s2_b):
    wid = lax.axis_index("s") * 2 + lax.axis_index("c")
    per_w = T // NW
    nch = per_w // COMB_CHUNK
    tbase = wid * per_w
    pltpu.sync_copy(dst1_hbm.at[pl.ds(tbase, per_w)], i1_v)
    pltpu.sync_copy(dst2_hbm.at[pl.ds(tbase, per_w)], i2_v)
    hs = (h_a, h_b)
    r1s = (r1_a, r1_b)
    r2s = (r2_a, r2_b)
    sems = ((sh_a, s1_a, s2_a), (sh_b, s1_b, s2_b))

    def start(c):
        p = c % 2
        t0 = tbase + c * COMB_CHUNK
        isl = pl.ds(c * COMB_CHUNK, COMB_CHUNK)
        return (
            pltpu.async_copy(h1_hbm.at[pl.ds(t0, COMB_CHUNK)], hs[p],
                             sems[p][0]),
            pltpu.async_copy(ybuf_hbm.at[i1_v.at[isl]], r1s[p], sems[p][1]),
            pltpu.async_copy(ybuf_hbm.at[i2_v.at[isl]], r2s[p], sems[p][2]),
        )

    pend = start(0)
    for c in range(nch):
        p = c % 2
        cur = pend
        if c + 1 < nch:
            pend = start(c + 1)
        for cp in cur:
            cp.wait()
        for i in range(COMB_CHUNK):
            def body(k, _):
                sl = pl.ds(k * 16, 16)
                hs[p][i, sl] = hs[p][i, sl] + r1s[p][i, sl] + r2s[p][i, sl]
                return 0
            lax.fori_loop(0, D // 16, body, 0)
        pltpu.sync_copy(hs[p],
                        out_hbm.at[pl.ds(tbase + c * COMB_CHUNK, COMB_CHUNK)])


@jax.jit
def kernel(hidden_states, ln1_w, ln2_w, router_w, w_gate, w_up, w_down):
    x = hidden_states.reshape(T, D)
    ln1 = ln1_w.reshape(1, D)
    ln2 = ln2_w.reshape(1, D)

    h1, xn, logits = pl.pallas_call(
        _norm_router_kernel,
        grid=(T // NORM_TB,),
        in_specs=[
            pl.BlockSpec((NORM_TB, D), lambda i: (i, 0)),
            pl.BlockSpec((1, D), lambda i: (0, 0)),
            pl.BlockSpec((1, D), lambda i: (0, 0)),
            pl.BlockSpec((D, E), lambda i: (0, 0)),
        ],
        out_specs=[
            pl.BlockSpec((NORM_TB, D), lambda i: (i, 0)),
            pl.BlockSpec((NORM_TB, D), lambda i: (i, 0)),
            pl.BlockSpec((NORM_TB, E), lambda i: (i, 0)),
        ],
        out_shape=[
            jax.ShapeDtypeStruct((T, D), jnp.float32),
            jax.ShapeDtypeStruct((T, D), jnp.float32),
            jax.ShapeDtypeStruct((T, E), jnp.float32),
        ],
    )(x, ln1, ln2, router_w)

    dst1_3d, dst2_3d, bexp = pl.pallas_call(
        _route_dst_kernel,
        grid=(T // RC,),
        in_specs=[
            pl.BlockSpec((T, E), lambda i: (0, 0)),
            pl.BlockSpec((RC, E), lambda i: (i, 0)),
        ],
        out_specs=[
            pl.BlockSpec((1, RC, 1), lambda i: (i, 0, 0)),
            pl.BlockSpec((1, RC, 1), lambda i: (i, 0, 0)),
            pl.BlockSpec((1, 1, NBLK), lambda i: (0, 0, 0)),
        ],
        out_shape=[
            jax.ShapeDtypeStruct((T // RC, RC, 1), jnp.int32),
            jax.ShapeDtypeStruct((T // RC, RC, 1), jnp.int32),
            jax.ShapeDtypeStruct((1, 1, NBLK), jnp.int32),
        ],
    )(logits, logits)

    perm_3d, wslot_3d = pl.pallas_call(
        _route_perm_kernel,
        grid=(NSLOT // PC,),
        in_specs=[
            pl.BlockSpec((T, E), lambda i: (0, 0)),
            pl.BlockSpec((T // RC, RC, 1), lambda i: (0, 0, 0)),
            pl.BlockSpec((T // RC, RC, 1), lambda i: (0, 0, 0)),
        ],
        out_specs=[
            pl.BlockSpec((1, 1, PC), lambda i: (i, 0, 0)),
            pl.BlockSpec((1, 1, PC), lambda i: (i, 0, 0)),
        ],
        out_shape=[
            jax.ShapeDtypeStruct((NSLOT // PC, 1, PC), jnp.int32),
            jax.ShapeDtypeStruct((NSLOT // PC, 1, PC), jnp.float32),
        ],
    )(logits, dst1_3d, dst2_3d)

    perm = perm_3d.reshape(NSLOT)
    dst1 = dst1_3d.reshape(T)
    dst2 = dst2_3d.reshape(T)
    bexp = bexp.reshape(NBLK)
    wslot2d = wslot_3d.reshape(NSLOT, 1)

    mesh = plsc.VectorSubcoreMesh(core_axis_name="c", subcore_axis_name="s")
    xbuf = pl.kernel(
        _dispatch_body,
        mesh=mesh,
        out_type=jax.ShapeDtypeStruct((NSLOT, D), jnp.float32),
        scratch_types=[
            pltpu.VMEM((NSLOT // NW,), jnp.int32),
            pltpu.VMEM((DISP_CHUNK, D), jnp.float32),
            pltpu.VMEM((DISP_CHUNK, D), jnp.float32),
            pltpu.SemaphoreType.DMA,
            pltpu.SemaphoreType.DMA,
        ],
    )(xn, perm)

    act = pl.pallas_call(
        _ffn_up_kernel,
        grid_spec=pltpu.PrefetchScalarGridSpec(
            num_scalar_prefetch=1,
            grid=(NBLK,),
            in_specs=[
                pl.BlockSpec((TB, D), lambda b, be: (b, 0)),
                pl.BlockSpec((1, D, F), lambda b, be: (be[b], 0, 0)),
                pl.BlockSpec((1, D, F), lambda b, be: (be[b], 0, 0)),
            ],
            out_specs=pl.BlockSpec((TB, F), lambda b, be: (b, 0)),
        ),
        out_shape=jax.ShapeDtypeStruct((NSLOT, F), jnp.float32),
        compiler_params=pltpu.CompilerParams(
            vmem_limit_bytes=60 * 1024 * 1024),
    )(bexp, xbuf, w_gate, w_up)

    ybuf = pl.pallas_call(
        _ffn_down_kernel,
        grid_spec=pltpu.PrefetchScalarGridSpec(
            num_scalar_prefetch=1,
            grid=(NBLK,),
            in_specs=[
                pl.BlockSpec((TB, F), lambda b, be: (b, 0)),
                pl.BlockSpec((TB, 1), lambda b, be: (b, 0)),
                pl.BlockSpec((1, F, D), lambda b, be: (be[b], 0, 0)),
            ],
            out_specs=pl.BlockSpec((TB, D), lambda b, be: (b, 0)),
        ),
        out_shape=jax.ShapeDtypeStruct((NSLOT, D), jnp.float32),
        compiler_params=pltpu.CompilerParams(
            vmem_limit_bytes=60 * 1024 * 1024),
    )(bexp, act, wslot2d, w_down)

    out = pl.kernel(
        _combine_body,
        mesh=mesh,
        out_type=jax.ShapeDtypeStruct((T, D), jnp.float32),
        scratch_types=(
            [pltpu.VMEM((T // NW,), jnp.int32)] * 2
            + [pltpu.VMEM((COMB_CHUNK, D), jnp.float32)] * 6
            + [pltpu.SemaphoreType.DMA] * 6
        ),
    )(h1, ybuf, dst1, dst2)

    return out.reshape(B, S, D)
